# Initial kernel scaffold; baseline (speedup 1.0000x reference)
import jax
import jax.numpy as jnp
from jax.experimental import pallas as pl


def kernel(x, z, W_meas_x, b_meas_x, W_vect_x, b_vect_x, W_meas_z, b_meas_z, W_vect_z, b_vect_z):
    raise NotImplementedError("write your pallas kernel here")


# speedup vs baseline: 10.1241x; 10.1241x over previous
import functools

import jax
import jax.numpy as jnp
from jax import lax
from jax.experimental import pallas as pl
from jax.experimental.pallas import tpu as pltpu
from jax.experimental.pallas import tpu_sc as plsc

_B = 8
_N = 2048
_D = 3
_K = 16
_DOUT = 128
_DMOM = 64
_C = _DOUT + _DMOM
_CP = 256
_RT = 256
_NT = _N // _RT

_NC, _NS = 2, 16
_NW = _NC * _NS
_PTS = _B * _N
_PPW = _PTS // _NW
_G = 8
_NGRP = _PPW // _G


def _tc_body(x_ref, xt_ref, wall_ref, wz_ref, ball_ref, z_ref,
             idx_ref, self_ref, nbr_ref):
    b = pl.program_id(0)
    xb = x_ref[0]
    xt = xt_ref[0]
    x0 = xb[:, 0:1]
    x1 = xb[:, 1:2]
    x2 = xb[:, 2:3]
    d0 = x0 - xt[0:1, :]
    d1 = x1 - xt[1:2, :]
    d2 = x2 - xt[2:3, :]
    c = d0 * d0 + d1 * d1 + d2 * d2

    wall = wall_ref[...]
    brow = ball_ref[...] + z_ref[0, 0] * wz_ref[...]
    self_ref[0] = x0 * wall[0:1, :] + x1 * wall[1:2, :] + x2 * wall[2:3, :] + brow
    pnbr = x0 * wall[3:4, :] + x1 * wall[4:5, :] + x2 * wall[5:6, :]
    nbr_ref[0] = jnp.concatenate(
        [pnbr, jnp.zeros((_RT, _CP - _C), jnp.float32)], axis=1)

    iota = lax.broadcasted_iota(jnp.int32, (_RT, _N), 1)
    cols = []
    for _ in range(_K):
        m = jnp.min(c, axis=1, keepdims=True)
        eq = c == m
        idxk = jnp.min(jnp.where(eq, iota, _N), axis=1, keepdims=True)
        c = jnp.where(iota == idxk, jnp.float32(jnp.inf), c)
        cols.append(idxk)
    idx_ref[0] = jnp.concatenate(cols, axis=1) + b * _N


def _tc_call(x, xt, wall, wz, ball, z3):
    return pl.pallas_call(
        _tc_body,
        grid=(_B, _NT),
        in_specs=[
            pl.BlockSpec((1, _RT, _D), lambda b, t: (b, t, 0)),
            pl.BlockSpec((1, _D, _N), lambda b, t: (b, 0, 0)),
            pl.BlockSpec((2 * _D, _C), lambda b, t: (0, 0)),
            pl.BlockSpec((1, _C), lambda b, t: (0, 0)),
            pl.BlockSpec((1, _C), lambda b, t: (0, 0)),
            pl.BlockSpec((1, 1, 1), lambda b, t: (b, 0, 0)),
        ],
        out_specs=[
            pl.BlockSpec((1, _RT, _K), lambda b, t: (b, t, 0)),
            pl.BlockSpec((1, _RT, _C), lambda b, t: (b, t, 0)),
            pl.BlockSpec((1, _RT, _CP), lambda b, t: (b, t, 0)),
        ],
        out_shape=[
            jax.ShapeDtypeStruct((_B, _N, _K), jnp.int32),
            jax.ShapeDtypeStruct((_B, _N, _C), jnp.float32),
            jax.ShapeDtypeStruct((_B, _N, _CP), jnp.float32),
        ],
    )(x, xt, wall, wz, ball, z3)


def _sc_body(nbr_hbm, self_hbm, idx_hbm, outx_hbm, outz_hbm,
             idx_v, rows_v, self_v, xstage_v, zacc_v, sem):
    wid = lax.axis_index("s") * _NC + lax.axis_index("c")
    base_pt = wid * _PPW

    for cc in range(_DMOM // 16):
        zacc_v[0, pl.ds(cc * 16, 16)] = jnp.zeros((16,), jnp.float32)

    def group(g, carry):
        pbase = base_pt + g * _G
        pltpu.sync_copy(idx_hbm.at[pl.ds(pbase * _K, _G * _K)], idx_v)
        pltpu.sync_copy(self_hbm.at[pl.ds(pbase, _G)], self_v)
        pltpu.async_copy(nbr_hbm.at[idx_v], rows_v, sem).wait()
        for p in range(_G):
            sx = [self_v[p, pl.ds(cc * 16, 16)] for cc in range(_C // 16)]

            def nbody(n, accs):
                out = []
                for cc in range(_C // 16):
                    v = rows_v[p * _K + n, pl.ds(cc * 16, 16)]
                    out.append(accs[cc] + jnp.maximum(sx[cc] + v, 0.0))
                return tuple(out)

            accs = lax.fori_loop(
                1, _K, nbody,
                tuple(jnp.zeros((16,), jnp.float32) for _ in range(_C // 16)))
            for cc in range(_DOUT // 16):
                xstage_v[p, pl.ds(cc * 16, 16)] = accs[cc] * (1.0 / (_K - 1))
            for cc in range(_DMOM // 16):
                j = pl.ds(cc * 16, 16)
                zacc_v[0, j] = zacc_v[0, j] + accs[_DOUT // 16 + cc]
        pltpu.sync_copy(xstage_v, outx_hbm.at[pl.ds(pbase, _G)])
        return carry

    lax.fori_loop(0, _NGRP, group, 0)
    pltpu.sync_copy(zacc_v, outz_hbm.at[pl.ds(wid, 1)])


@functools.cache
def _sc_call():
    return pl.kernel(
        _sc_body,
        out_type=[
            jax.ShapeDtypeStruct((_PTS, _DOUT), jnp.float32),
            jax.ShapeDtypeStruct((_NW, _DMOM), jnp.float32),
        ],
        mesh=plsc.VectorSubcoreMesh(core_axis_name="c", subcore_axis_name="s"),
        scratch_types=[
            pltpu.VMEM((_G * _K,), jnp.int32),
            pltpu.VMEM((_G * _K, _CP), jnp.float32),
            pltpu.VMEM((_G, _C), jnp.float32),
            pltpu.VMEM((_G, _DOUT), jnp.float32),
            pltpu.VMEM((1, _DMOM), jnp.float32),
            pltpu.SemaphoreType.DMA,
        ],
    )


def kernel(x, z, W_meas_x, b_meas_x, W_vect_x, b_vect_x,
           W_meas_z, b_meas_z, W_vect_z, b_vect_z):
    wall = jnp.concatenate([W_meas_x.T, W_vect_x.T], axis=1)
    wz = jnp.concatenate([W_meas_z.T, W_vect_z.T], axis=1)
    ball = jnp.concatenate([b_meas_x + b_meas_z,
                            b_vect_x + b_vect_z])[None, :]
    xt = x.transpose(0, 2, 1)
    z3 = z.reshape(_B, 1, 1)

    idx, selft, nbrt = _tc_call(x, xt, wall, wz, ball, z3)
    outx, outz = _sc_call()(
        nbrt.reshape(_PTS, _CP),
        selft.reshape(_PTS, _C),
        idx.reshape(_PTS * _K),
    )

    x_new = outx.reshape(_B, _N * _DOUT)
    z_new = outz.reshape(_B, _NW // _B, _DMOM).sum(axis=1) * (
        1.0 / ((_K - 1) * _N))
    return (x_new, z_new)


# speedup vs baseline: 12.0704x; 1.1922x over previous
import functools

import jax
import jax.numpy as jnp
from jax import lax
from jax.experimental import pallas as pl
from jax.experimental.pallas import tpu as pltpu
from jax.experimental.pallas import tpu_sc as plsc

_B = 8
_N = 2048
_D = 3
_K = 16
_DOUT = 128
_DMOM = 64
_C = _DOUT + _DMOM
_CP = 256
_RT = 256
_NT = _N // _RT

_NC, _NS = 2, 16
_NW = _NC * _NS
_PTS = _B * _N
_PPW = _PTS // _NW
_G = 8
_NGRP = _PPW // _G


def _tc_body(x_ref, xt_ref, wall_ref, wz_ref, ball_ref, z_ref,
             idx_ref, self_ref, nbr_ref):
    b = pl.program_id(0)
    xb = x_ref[0]
    xt = xt_ref[0]
    x0 = xb[:, 0:1]
    x1 = xb[:, 1:2]
    x2 = xb[:, 2:3]
    xt0 = xt[0:1, :]
    xt1 = xt[1:2, :]
    xt2 = xt[2:3, :]
    x2r = x0 * x0 + x1 * x1 + x2 * x2
    x2c = xt0 * xt0 + xt1 * xt1 + xt2 * xt2
    dot = jnp.dot(xb, xt, preferred_element_type=jnp.float32)
    c = jnp.maximum(x2r + x2c - 2.0 * dot, 0.0)

    wall = wall_ref[...]
    brow = ball_ref[...] + z_ref[0, 0] * wz_ref[...]
    self_ref[0] = x0 * wall[0:1, :] + x1 * wall[1:2, :] + x2 * wall[2:3, :] + brow
    pnbr = x0 * wall[3:4, :] + x1 * wall[4:5, :] + x2 * wall[5:6, :]
    nbr_ref[0] = jnp.concatenate(
        [pnbr, jnp.zeros((_RT, _CP - _C), jnp.float32)], axis=1)

    t = pl.program_id(1)
    iota = lax.broadcasted_iota(jnp.int32, (_RT, _N), 1)
    riota = lax.broadcasted_iota(jnp.int32, (_RT, 1), 0) + t * _RT
    c = jnp.where(iota == riota, jnp.float32(jnp.inf), c)
    cols = [riota]
    for _ in range(_K - 1):
        m = jnp.min(c, axis=1, keepdims=True)
        eq = c == m
        idxk = jnp.min(jnp.where(eq, iota, _N), axis=1, keepdims=True)
        c = jnp.where(eq, jnp.float32(jnp.inf), c)
        cols.append(idxk)
    idx_ref[0] = jnp.concatenate(cols, axis=1) + b * _N


def _tc_call(x, xt, wall, wz, ball, z3):
    return pl.pallas_call(
        _tc_body,
        grid=(_B, _NT),
        in_specs=[
            pl.BlockSpec((1, _RT, _D), lambda b, t: (b, t, 0)),
            pl.BlockSpec((1, _D, _N), lambda b, t: (b, 0, 0)),
            pl.BlockSpec((2 * _D, _C), lambda b, t: (0, 0)),
            pl.BlockSpec((1, _C), lambda b, t: (0, 0)),
            pl.BlockSpec((1, _C), lambda b, t: (0, 0)),
            pl.BlockSpec((1, 1, 1), lambda b, t: (b, 0, 0)),
        ],
        out_specs=[
            pl.BlockSpec((1, _RT, _K), lambda b, t: (b, t, 0)),
            pl.BlockSpec((1, _RT, _C), lambda b, t: (b, t, 0)),
            pl.BlockSpec((1, _RT, _CP), lambda b, t: (b, t, 0)),
        ],
        out_shape=[
            jax.ShapeDtypeStruct((_B, _N, _K), jnp.int32),
            jax.ShapeDtypeStruct((_B, _N, _C), jnp.float32),
            jax.ShapeDtypeStruct((_B, _N, _CP), jnp.float32),
        ],
    )(x, xt, wall, wz, ball, z3)


def _sc_body(nbr_hbm, self_hbm, idx_hbm, outx_hbm, outz_hbm,
             idx_v, rows_v, self_v, xstage_v, zacc_v, sem):
    wid = lax.axis_index("s") * _NC + lax.axis_index("c")
    base_pt = wid * _PPW

    for cc in range(_DMOM // 16):
        zacc_v[0, pl.ds(cc * 16, 16)] = jnp.zeros((16,), jnp.float32)

    def group(g, carry):
        pbase = base_pt + g * _G
        pltpu.sync_copy(idx_hbm.at[pl.ds(pbase * _K, _G * _K)], idx_v)
        pltpu.sync_copy(self_hbm.at[pl.ds(pbase, _G)], self_v)
        pltpu.async_copy(nbr_hbm.at[idx_v], rows_v, sem).wait()
        for p in range(_G):
            sx = [self_v[p, pl.ds(cc * 16, 16)] for cc in range(_C // 16)]

            def nbody(n, accs):
                out = []
                for cc in range(_C // 16):
                    v = rows_v[p * _K + n, pl.ds(cc * 16, 16)]
                    out.append(accs[cc] + jnp.maximum(sx[cc] + v, 0.0))
                return tuple(out)

            accs = lax.fori_loop(
                1, _K, nbody,
                tuple(jnp.zeros((16,), jnp.float32) for _ in range(_C // 16)))
            for cc in range(_DOUT // 16):
                xstage_v[p, pl.ds(cc * 16, 16)] = accs[cc] * (1.0 / (_K - 1))
            for cc in range(_DMOM // 16):
                j = pl.ds(cc * 16, 16)
                zacc_v[0, j] = zacc_v[0, j] + accs[_DOUT // 16 + cc]
        pltpu.sync_copy(xstage_v, outx_hbm.at[pl.ds(pbase, _G)])
        return carry

    lax.fori_loop(0, _NGRP, group, 0)
    pltpu.sync_copy(zacc_v, outz_hbm.at[pl.ds(wid, 1)])


@functools.cache
def _sc_call():
    return pl.kernel(
        _sc_body,
        out_type=[
            jax.ShapeDtypeStruct((_PTS, _DOUT), jnp.float32),
            jax.ShapeDtypeStruct((_NW, _DMOM), jnp.float32),
        ],
        mesh=plsc.VectorSubcoreMesh(core_axis_name="c", subcore_axis_name="s"),
        scratch_types=[
            pltpu.VMEM((_G * _K,), jnp.int32),
            pltpu.VMEM((_G * _K, _CP), jnp.float32),
            pltpu.VMEM((_G, _C), jnp.float32),
            pltpu.VMEM((_G, _DOUT), jnp.float32),
            pltpu.VMEM((1, _DMOM), jnp.float32),
            pltpu.SemaphoreType.DMA,
        ],
    )


def kernel(x, z, W_meas_x, b_meas_x, W_vect_x, b_vect_x,
           W_meas_z, b_meas_z, W_vect_z, b_vect_z):
    wall = jnp.concatenate([W_meas_x.T, W_vect_x.T], axis=1)
    wz = jnp.concatenate([W_meas_z.T, W_vect_z.T], axis=1)
    ball = jnp.concatenate([b_meas_x + b_meas_z,
                            b_vect_x + b_vect_z])[None, :]
    xt = x.transpose(0, 2, 1)
    z3 = z.reshape(_B, 1, 1)

    idx, selft, nbrt = _tc_call(x, xt, wall, wz, ball, z3)
    outx, outz = _sc_call()(
        nbrt.reshape(_PTS, _CP),
        selft.reshape(_PTS, _C),
        idx.reshape(_PTS * _K),
    )

    x_new = outx.reshape(_B, _N * _DOUT)
    z_new = outz.reshape(_B, _NW // _B, _DMOM).sum(axis=1) * (
        1.0 / ((_K - 1) * _N))
    return (x_new, z_new)


# speedup vs baseline: 14.3663x; 1.1902x over previous
import functools

import jax
import jax.numpy as jnp
from jax import lax
from jax.experimental import pallas as pl
from jax.experimental.pallas import tpu as pltpu
from jax.experimental.pallas import tpu_sc as plsc

_B = 8
_N = 2048
_D = 3
_K = 16
_DOUT = 128
_DMOM = 64
_C = _DOUT + _DMOM
_CP = 256
_RT = 256
_NT = _N // _RT

_NC, _NS = 2, 16
_NW = _NC * _NS
_BH = _B // 2
_PTS = _BH * _N
_PPW = _PTS // _NW
_G = 8
_NGRP = _PPW // _G


def _tc_body(x_ref, xt_ref, wall_ref, wz_ref, ball_ref, z_ref,
             idx_ref, self_ref, nbr_ref):
    b = pl.program_id(0)
    xb = x_ref[0]
    xt = xt_ref[0]
    x0 = xb[:, 0:1]
    x1 = xb[:, 1:2]
    x2 = xb[:, 2:3]
    xt0 = xt[0:1, :]
    xt1 = xt[1:2, :]
    xt2 = xt[2:3, :]
    x2r = x0 * x0 + x1 * x1 + x2 * x2
    x2c = xt0 * xt0 + xt1 * xt1 + xt2 * xt2
    dot = jnp.dot(xb, xt, preferred_element_type=jnp.float32)
    c = jnp.maximum(x2r + x2c - 2.0 * dot, 0.0)

    wall = wall_ref[...]
    brow = ball_ref[...] + z_ref[0, 0] * wz_ref[...]
    self_ref[0] = x0 * wall[0:1, :] + x1 * wall[1:2, :] + x2 * wall[2:3, :] + brow
    pnbr = x0 * wall[3:4, :] + x1 * wall[4:5, :] + x2 * wall[5:6, :]
    nbr_ref[0] = jnp.concatenate(
        [pnbr, jnp.zeros((_RT, _CP - _C), jnp.float32)], axis=1)

    t = pl.program_id(1)
    iota = lax.broadcasted_iota(jnp.int32, (_RT, _N), 1)
    riota = lax.broadcasted_iota(jnp.int32, (_RT, 1), 0) + t * _RT
    c = jnp.where(iota == riota, jnp.float32(jnp.inf), c)
    cols = [riota]
    for _ in range(_K - 1):
        m = jnp.min(c, axis=1, keepdims=True)
        eq = c == m
        idxk = jnp.min(jnp.where(eq, iota, _N), axis=1, keepdims=True)
        c = jnp.where(eq, jnp.float32(jnp.inf), c)
        cols.append(idxk)
    idx_ref[0] = jnp.concatenate(cols, axis=1) + b * _N


def _tc_call(x, xt, wall, wz, ball, z3):
    return pl.pallas_call(
        _tc_body,
        grid=(_BH, _NT),
        in_specs=[
            pl.BlockSpec((1, _RT, _D), lambda b, t: (b, t, 0)),
            pl.BlockSpec((1, _D, _N), lambda b, t: (b, 0, 0)),
            pl.BlockSpec((2 * _D, _C), lambda b, t: (0, 0)),
            pl.BlockSpec((1, _C), lambda b, t: (0, 0)),
            pl.BlockSpec((1, _C), lambda b, t: (0, 0)),
            pl.BlockSpec((1, 1, 1), lambda b, t: (b, 0, 0)),
        ],
        out_specs=[
            pl.BlockSpec((1, _RT, _K), lambda b, t: (b, t, 0)),
            pl.BlockSpec((1, _RT, _C), lambda b, t: (b, t, 0)),
            pl.BlockSpec((1, _RT, _CP), lambda b, t: (b, t, 0)),
        ],
        out_shape=[
            jax.ShapeDtypeStruct((_BH, _N, _K), jnp.int32),
            jax.ShapeDtypeStruct((_BH, _N, _C), jnp.float32),
            jax.ShapeDtypeStruct((_BH, _N, _CP), jnp.float32),
        ],
    )(x, xt, wall, wz, ball, z3)


def _sc_body(nbr_hbm, self_hbm, idx_hbm, outx_hbm, outz_hbm,
             idx_v, rows_v, self_v, xstage_v, zacc_v, sem):
    wid = lax.axis_index("s") * _NC + lax.axis_index("c")
    base_pt = wid * _PPW

    for cc in range(_DMOM // 16):
        zacc_v[0, pl.ds(cc * 16, 16)] = jnp.zeros((16,), jnp.float32)

    def group(g, carry):
        pbase = base_pt + g * _G
        pltpu.sync_copy(idx_hbm.at[pl.ds(pbase * _K, _G * _K)], idx_v)
        pltpu.sync_copy(self_hbm.at[pl.ds(pbase, _G)], self_v)
        pltpu.async_copy(nbr_hbm.at[idx_v], rows_v, sem).wait()
        for p in range(_G):
            sx = [self_v[p, pl.ds(cc * 16, 16)] for cc in range(_C // 16)]

            def nbody(n, accs):
                out = []
                for cc in range(_C // 16):
                    v = rows_v[p * _K + n, pl.ds(cc * 16, 16)]
                    out.append(accs[cc] + jnp.maximum(sx[cc] + v, 0.0))
                return tuple(out)

            accs = lax.fori_loop(
                1, _K, nbody,
                tuple(jnp.zeros((16,), jnp.float32) for _ in range(_C // 16)))
            for cc in range(_DOUT // 16):
                xstage_v[p, pl.ds(cc * 16, 16)] = accs[cc] * (1.0 / (_K - 1))
            for cc in range(_DMOM // 16):
                j = pl.ds(cc * 16, 16)
                zacc_v[0, j] = zacc_v[0, j] + accs[_DOUT // 16 + cc]
        pltpu.sync_copy(xstage_v, outx_hbm.at[pl.ds(pbase, _G)])
        return carry

    lax.fori_loop(0, _NGRP, group, 0)
    pltpu.sync_copy(zacc_v, outz_hbm.at[pl.ds(wid, 1)])


@functools.cache
def _sc_call():
    return pl.kernel(
        _sc_body,
        out_type=[
            jax.ShapeDtypeStruct((_PTS, _DOUT), jnp.float32),
            jax.ShapeDtypeStruct((_NW, _DMOM), jnp.float32),
        ],
        mesh=plsc.VectorSubcoreMesh(core_axis_name="c", subcore_axis_name="s"),
        scratch_types=[
            pltpu.VMEM((_G * _K,), jnp.int32),
            pltpu.VMEM((_G * _K, _CP), jnp.float32),
            pltpu.VMEM((_G, _C), jnp.float32),
            pltpu.VMEM((_G, _DOUT), jnp.float32),
            pltpu.VMEM((1, _DMOM), jnp.float32),
            pltpu.SemaphoreType.DMA,
        ],
    )


def kernel(x, z, W_meas_x, b_meas_x, W_vect_x, b_vect_x,
           W_meas_z, b_meas_z, W_vect_z, b_vect_z):
    wall = jnp.concatenate([W_meas_x.T, W_vect_x.T], axis=1)
    wz = jnp.concatenate([W_meas_z.T, W_vect_z.T], axis=1)
    ball = jnp.concatenate([b_meas_x + b_meas_z,
                            b_vect_x + b_vect_z])[None, :]
    xt = x.transpose(0, 2, 1)
    z3 = z.reshape(_B, 1, 1)

    tc_half = [
        _tc_call(x[h * _BH:(h + 1) * _BH], xt[h * _BH:(h + 1) * _BH],
                 wall, wz, ball, z3[h * _BH:(h + 1) * _BH])
        for h in range(2)
    ]
    sc_half = [
        _sc_call()(
            nbrt.reshape(_PTS, _CP),
            selft.reshape(_PTS, _C),
            idx.reshape(_PTS * _K),
        )
        for idx, selft, nbrt in tc_half
    ]

    x_new = jnp.concatenate(
        [outx.reshape(_BH, _N * _DOUT) for outx, _ in sc_half], axis=0)
    z_new = jnp.concatenate(
        [outz.reshape(_BH, _NW // _BH, _DMOM).sum(axis=1)
         for _, outz in sc_half], axis=0) * (1.0 / ((_K - 1) * _N))
    return (x_new, z_new)


# speedup vs baseline: 15.3961x; 1.0717x over previous
import functools

import jax
import jax.numpy as jnp
from jax import lax
from jax.experimental import pallas as pl
from jax.experimental.pallas import tpu as pltpu
from jax.experimental.pallas import tpu_sc as plsc

_B = 8
_N = 2048
_D = 3
_K = 16
_DOUT = 128
_DMOM = 64
_C = _DOUT + _DMOM
_CP = 256
_RT = 256
_NT = _N // _RT

_NC, _NS = 2, 16
_NW = _NC * _NS
_BH = _B // 4
_PTS = _BH * _N
_PPW = _PTS // _NW
_G = 8
_NGRP = _PPW // _G


def _tc_body(x_ref, xt_ref, wall_ref, wz_ref, ball_ref, z_ref,
             idx_ref, self_ref, nbr_ref):
    b = pl.program_id(0)
    xb = x_ref[0]
    xt = xt_ref[0]
    x0 = xb[:, 0:1]
    x1 = xb[:, 1:2]
    x2 = xb[:, 2:3]
    xt0 = xt[0:1, :]
    xt1 = xt[1:2, :]
    xt2 = xt[2:3, :]
    x2r = x0 * x0 + x1 * x1 + x2 * x2
    x2c = xt0 * xt0 + xt1 * xt1 + xt2 * xt2
    dot = jnp.dot(xb, xt, preferred_element_type=jnp.float32)
    c = jnp.maximum(x2r + x2c - 2.0 * dot, 0.0)

    wall = wall_ref[...]
    brow = ball_ref[...] + z_ref[0, 0] * wz_ref[...]
    self_ref[0] = x0 * wall[0:1, :] + x1 * wall[1:2, :] + x2 * wall[2:3, :] + brow
    pnbr = x0 * wall[3:4, :] + x1 * wall[4:5, :] + x2 * wall[5:6, :]
    nbr_ref[0] = jnp.concatenate(
        [pnbr, jnp.zeros((_RT, _CP - _C), jnp.float32)], axis=1)

    t = pl.program_id(1)
    iota = lax.broadcasted_iota(jnp.int32, (_RT, _N), 1)
    riota = lax.broadcasted_iota(jnp.int32, (_RT, 1), 0) + t * _RT
    c = jnp.where(iota == riota, jnp.float32(jnp.inf), c)
    cols = [riota]
    for _ in range(_K - 1):
        m = jnp.min(c, axis=1, keepdims=True)
        eq = c == m
        idxk = jnp.min(jnp.where(eq, iota, _N), axis=1, keepdims=True)
        c = jnp.where(eq, jnp.float32(jnp.inf), c)
        cols.append(idxk)
    idx_ref[0] = jnp.concatenate(cols, axis=1) + b * _N


def _tc_call(x, xt, wall, wz, ball, z3):
    return pl.pallas_call(
        _tc_body,
        grid=(_BH, _NT),
        in_specs=[
            pl.BlockSpec((1, _RT, _D), lambda b, t: (b, t, 0)),
            pl.BlockSpec((1, _D, _N), lambda b, t: (b, 0, 0)),
            pl.BlockSpec((2 * _D, _C), lambda b, t: (0, 0)),
            pl.BlockSpec((1, _C), lambda b, t: (0, 0)),
            pl.BlockSpec((1, _C), lambda b, t: (0, 0)),
            pl.BlockSpec((1, 1, 1), lambda b, t: (b, 0, 0)),
        ],
        out_specs=[
            pl.BlockSpec((1, _RT, _K), lambda b, t: (b, t, 0)),
            pl.BlockSpec((1, _RT, _C), lambda b, t: (b, t, 0)),
            pl.BlockSpec((1, _RT, _CP), lambda b, t: (b, t, 0)),
        ],
        out_shape=[
            jax.ShapeDtypeStruct((_BH, _N, _K), jnp.int32),
            jax.ShapeDtypeStruct((_BH, _N, _C), jnp.float32),
            jax.ShapeDtypeStruct((_BH, _N, _CP), jnp.float32),
        ],
    )(x, xt, wall, wz, ball, z3)


def _sc_body(nbr_hbm, self_hbm, idx_hbm, outx_hbm, outz_hbm,
             idx_v, rows_v, self_v, xstage_v, zacc_v, sem):
    wid = lax.axis_index("s") * _NC + lax.axis_index("c")
    base_pt = wid * _PPW

    for cc in range(_DMOM // 16):
        zacc_v[0, pl.ds(cc * 16, 16)] = jnp.zeros((16,), jnp.float32)

    def group(g, carry):
        pbase = base_pt + g * _G
        pltpu.sync_copy(idx_hbm.at[pl.ds(pbase * _K, _G * _K)], idx_v)
        pltpu.sync_copy(self_hbm.at[pl.ds(pbase, _G)], self_v)
        pltpu.async_copy(nbr_hbm.at[idx_v], rows_v, sem).wait()
        for p in range(_G):
            sx = [self_v[p, pl.ds(cc * 16, 16)] for cc in range(_C // 16)]

            def nbody(n, accs):
                out = []
                for cc in range(_C // 16):
                    v = rows_v[p * _K + n, pl.ds(cc * 16, 16)]
                    out.append(accs[cc] + jnp.maximum(sx[cc] + v, 0.0))
                return tuple(out)

            accs = lax.fori_loop(
                1, _K, nbody,
                tuple(jnp.zeros((16,), jnp.float32) for _ in range(_C // 16)))
            for cc in range(_DOUT // 16):
                xstage_v[p, pl.ds(cc * 16, 16)] = accs[cc] * (1.0 / (_K - 1))
            for cc in range(_DMOM // 16):
                j = pl.ds(cc * 16, 16)
                zacc_v[0, j] = zacc_v[0, j] + accs[_DOUT // 16 + cc]
        pltpu.sync_copy(xstage_v, outx_hbm.at[pl.ds(pbase, _G)])
        return carry

    lax.fori_loop(0, _NGRP, group, 0)
    pltpu.sync_copy(zacc_v, outz_hbm.at[pl.ds(wid, 1)])


@functools.cache
def _sc_call():
    return pl.kernel(
        _sc_body,
        out_type=[
            jax.ShapeDtypeStruct((_PTS, _DOUT), jnp.float32),
            jax.ShapeDtypeStruct((_NW, _DMOM), jnp.float32),
        ],
        mesh=plsc.VectorSubcoreMesh(core_axis_name="c", subcore_axis_name="s"),
        scratch_types=[
            pltpu.VMEM((_G * _K,), jnp.int32),
            pltpu.VMEM((_G * _K, _CP), jnp.float32),
            pltpu.VMEM((_G, _C), jnp.float32),
            pltpu.VMEM((_G, _DOUT), jnp.float32),
            pltpu.VMEM((1, _DMOM), jnp.float32),
            pltpu.SemaphoreType.DMA,
        ],
    )


def kernel(x, z, W_meas_x, b_meas_x, W_vect_x, b_vect_x,
           W_meas_z, b_meas_z, W_vect_z, b_vect_z):
    wall = jnp.concatenate([W_meas_x.T, W_vect_x.T], axis=1)
    wz = jnp.concatenate([W_meas_z.T, W_vect_z.T], axis=1)
    ball = jnp.concatenate([b_meas_x + b_meas_z,
                            b_vect_x + b_vect_z])[None, :]
    xt = x.transpose(0, 2, 1)
    z3 = z.reshape(_B, 1, 1)

    tc_half = [
        _tc_call(x[h * _BH:(h + 1) * _BH], xt[h * _BH:(h + 1) * _BH],
                 wall, wz, ball, z3[h * _BH:(h + 1) * _BH])
        for h in range(_B // _BH)
    ]
    sc_half = [
        _sc_call()(
            nbrt.reshape(_PTS, _CP),
            selft.reshape(_PTS, _C),
            idx.reshape(_PTS * _K),
        )
        for idx, selft, nbrt in tc_half
    ]

    x_new = jnp.concatenate(
        [outx.reshape(_BH, _N * _DOUT) for outx, _ in sc_half], axis=0)
    z_new = jnp.concatenate(
        [outz.reshape(_BH, _NW // _BH, _DMOM).sum(axis=1)
         for _, outz in sc_half], axis=0) * (1.0 / ((_K - 1) * _N))
    return (x_new, z_new)


# speedup vs baseline: 16.2976x; 1.0586x over previous
import functools

import jax
import jax.numpy as jnp
from jax import lax
from jax.experimental import pallas as pl
from jax.experimental.pallas import tpu as pltpu
from jax.experimental.pallas import tpu_sc as plsc

_B = 8
_N = 2048
_D = 3
_K = 16
_DOUT = 128
_DMOM = 64
_C = _DOUT + _DMOM
_CP = 256
_RT = 256
_NT = _N // _RT

_NC, _NS = 2, 16
_NW = _NC * _NS
_BH = _B // 4
_PTS = _BH * _N
_PPW = _PTS // _NW
_G = 8
_NGRP = _PPW // _G


def _tc_body(x_ref, xt_ref, wall_ref, wz_ref, ball_ref, z_ref,
             idx_ref, self_ref, nbr_ref):
    b = pl.program_id(0)
    xb = x_ref[0]
    xt = xt_ref[0]
    x0 = xb[:, 0:1]
    x1 = xb[:, 1:2]
    x2 = xb[:, 2:3]
    xt0 = xt[0:1, :]
    xt1 = xt[1:2, :]
    xt2 = xt[2:3, :]
    x2r = x0 * x0 + x1 * x1 + x2 * x2
    x2c = xt0 * xt0 + xt1 * xt1 + xt2 * xt2
    dot = jnp.dot(xb, xt, preferred_element_type=jnp.float32)
    c = jnp.maximum(x2r + x2c - 2.0 * dot, 0.0)

    wall = wall_ref[...]
    brow = ball_ref[...] + z_ref[0, 0] * wz_ref[...]
    self_ref[0] = x0 * wall[0:1, :] + x1 * wall[1:2, :] + x2 * wall[2:3, :] + brow
    pnbr = x0 * wall[3:4, :] + x1 * wall[4:5, :] + x2 * wall[5:6, :]
    nbr_ref[0] = jnp.concatenate(
        [pnbr, jnp.zeros((_RT, _CP - _C), jnp.float32)], axis=1)

    t = pl.program_id(1)
    iota = lax.broadcasted_iota(jnp.int32, (_RT, _N), 1)
    riota = lax.broadcasted_iota(jnp.int32, (_RT, 1), 0) + t * _RT
    c = jnp.where(iota == riota, jnp.float32(jnp.inf), c)
    iotacol = lax.broadcasted_iota(jnp.int32, (_N, 1), 0).astype(jnp.float32)
    cols = [riota]
    for _ in range(_K - 1):
        m = jnp.min(c, axis=1, keepdims=True)
        eq = c == m
        idxf = jnp.dot(eq.astype(jnp.float32), iotacol,
                       preferred_element_type=jnp.float32)
        idxk = jnp.minimum(idxf, _N - 1).astype(jnp.int32)
        c = jnp.where(eq, jnp.float32(jnp.inf), c)
        cols.append(idxk)
    idx_ref[0] = jnp.concatenate(cols, axis=1) + b * _N


def _tc_call(x, xt, wall, wz, ball, z3):
    return pl.pallas_call(
        _tc_body,
        grid=(_BH, _NT),
        in_specs=[
            pl.BlockSpec((1, _RT, _D), lambda b, t: (b, t, 0)),
            pl.BlockSpec((1, _D, _N), lambda b, t: (b, 0, 0)),
            pl.BlockSpec((2 * _D, _C), lambda b, t: (0, 0)),
            pl.BlockSpec((1, _C), lambda b, t: (0, 0)),
            pl.BlockSpec((1, _C), lambda b, t: (0, 0)),
            pl.BlockSpec((1, 1, 1), lambda b, t: (b, 0, 0)),
        ],
        out_specs=[
            pl.BlockSpec((1, _RT, _K), lambda b, t: (b, t, 0)),
            pl.BlockSpec((1, _RT, _C), lambda b, t: (b, t, 0)),
            pl.BlockSpec((1, _RT, _CP), lambda b, t: (b, t, 0)),
        ],
        out_shape=[
            jax.ShapeDtypeStruct((_BH, _N, _K), jnp.int32),
            jax.ShapeDtypeStruct((_BH, _N, _C), jnp.float32),
            jax.ShapeDtypeStruct((_BH, _N, _CP), jnp.float32),
        ],
    )(x, xt, wall, wz, ball, z3)


def _sc_body(nbr_hbm, self_hbm, idx_hbm, outx_hbm, outz_hbm,
             idx_v, rows_v, self_v, xstage_v, zacc_v, sem):
    wid = lax.axis_index("s") * _NC + lax.axis_index("c")
    base_pt = wid * _PPW

    for cc in range(_DMOM // 16):
        zacc_v[0, pl.ds(cc * 16, 16)] = jnp.zeros((16,), jnp.float32)

    def group(g, carry):
        pbase = base_pt + g * _G
        pltpu.sync_copy(idx_hbm.at[pl.ds(pbase * _K, _G * _K)], idx_v)
        pltpu.sync_copy(self_hbm.at[pl.ds(pbase, _G)], self_v)
        pltpu.async_copy(nbr_hbm.at[idx_v], rows_v, sem).wait()
        for p in range(_G):
            sx = [self_v[p, pl.ds(cc * 16, 16)] for cc in range(_C // 16)]

            def nbody(n, accs):
                out = []
                for cc in range(_C // 16):
                    v = rows_v[p * _K + n, pl.ds(cc * 16, 16)]
                    out.append(accs[cc] + jnp.maximum(sx[cc] + v, 0.0))
                return tuple(out)

            accs = lax.fori_loop(
                1, _K, nbody,
                tuple(jnp.zeros((16,), jnp.float32) for _ in range(_C // 16)))
            for cc in range(_DOUT // 16):
                xstage_v[p, pl.ds(cc * 16, 16)] = accs[cc] * (1.0 / (_K - 1))
            for cc in range(_DMOM // 16):
                j = pl.ds(cc * 16, 16)
                zacc_v[0, j] = zacc_v[0, j] + accs[_DOUT // 16 + cc]
        pltpu.sync_copy(xstage_v, outx_hbm.at[pl.ds(pbase, _G)])
        return carry

    lax.fori_loop(0, _NGRP, group, 0)
    pltpu.sync_copy(zacc_v, outz_hbm.at[pl.ds(wid, 1)])


@functools.cache
def _sc_call():
    return pl.kernel(
        _sc_body,
        out_type=[
            jax.ShapeDtypeStruct((_PTS, _DOUT), jnp.float32),
            jax.ShapeDtypeStruct((_NW, _DMOM), jnp.float32),
        ],
        mesh=plsc.VectorSubcoreMesh(core_axis_name="c", subcore_axis_name="s"),
        scratch_types=[
            pltpu.VMEM((_G * _K,), jnp.int32),
            pltpu.VMEM((_G * _K, _CP), jnp.float32),
            pltpu.VMEM((_G, _C), jnp.float32),
            pltpu.VMEM((_G, _DOUT), jnp.float32),
            pltpu.VMEM((1, _DMOM), jnp.float32),
            pltpu.SemaphoreType.DMA,
        ],
    )


def kernel(x, z, W_meas_x, b_meas_x, W_vect_x, b_vect_x,
           W_meas_z, b_meas_z, W_vect_z, b_vect_z):
    wall = jnp.concatenate([W_meas_x.T, W_vect_x.T], axis=1)
    wz = jnp.concatenate([W_meas_z.T, W_vect_z.T], axis=1)
    ball = jnp.concatenate([b_meas_x + b_meas_z,
                            b_vect_x + b_vect_z])[None, :]
    xt = x.transpose(0, 2, 1)
    z3 = z.reshape(_B, 1, 1)

    tc_half = [
        _tc_call(x[h * _BH:(h + 1) * _BH], xt[h * _BH:(h + 1) * _BH],
                 wall, wz, ball, z3[h * _BH:(h + 1) * _BH])
        for h in range(_B // _BH)
    ]
    sc_half = [
        _sc_call()(
            nbrt.reshape(_PTS, _CP),
            selft.reshape(_PTS, _C),
            idx.reshape(_PTS * _K),
        )
        for idx, selft, nbrt in tc_half
    ]

    x_new = jnp.concatenate(
        [outx.reshape(_BH, _N * _DOUT) for outx, _ in sc_half], axis=0)
    z_new = jnp.concatenate(
        [outz.reshape(_BH, _NW // _BH, _DMOM).sum(axis=1)
         for _, outz in sc_half], axis=0) * (1.0 / ((_K - 1) * _N))
    return (x_new, z_new)
